# baseline (speedup 1.0000x reference)
import functools

import jax
import jax.numpy as jnp
from jax import lax
from jax.experimental import pallas as pl
from jax.experimental.pallas import tpu as pltpu
from jax.experimental.pallas import tpu_sc as plsc

_N = 10000
_NC = 2
_NS = 16
_NW = _NC * _NS
_L = 16


@functools.lru_cache(maxsize=None)
def _make_spmm(n_edges: int, d: int, chunk: int, use_bf16: bool = False,
               paired_out: bool = False, dual_src: bool = False):
    ew = n_edges // _NW
    assert ew * _NW == n_edges and ew % chunk == 0
    assert chunk % 8 == 0 and chunk <= 128 and d % _L == 0
    vdt = jnp.bfloat16 if use_bf16 else jnp.float32
    vl = 2 * _L if use_bf16 else _L
    dsrc = 2 * d if dual_src else d
    assert not dual_src or (use_bf16 and d == 2 * vl)
    nchunks = ew // chunk
    zr = chunk
    nzc = _N // zr
    npers = -(-nzc // _NS)
    assert _N % zr == 0
    unroll = 8
    nb = 4
    la = 2
    assert nchunks >= nb
    assert chunk % unroll == 0
    mesh = plsc.VectorSubcoreMesh(
        core_axis_name="c", subcore_axis_name="s",
        num_cores=_NC, num_subcores=_NS)

    out_shape = (_N, _NC, d) if paired_out else (_NC, _N, d)

    @functools.partial(
        pl.kernel,
        out_type=jax.ShapeDtypeStruct(out_shape, vdt),
        mesh=mesh,
        compiler_params=pltpu.CompilerParams(
            needs_layout_passes=False, use_tc_tiling_on_sc=False),
        scratch_types=[
            pltpu.VMEM((ew,), jnp.int32),
            pltpu.VMEM((ew,), jnp.int32),
            pltpu.VMEM((ew,), jnp.float32),
            [pltpu.VMEM((chunk, dsrc), vdt)] * nb,
            ([pltpu.VMEM((chunk, d), vdt)] * nb
             if dual_src else pltpu.SMEM((1,), jnp.int32)),
            pltpu.VMEM((64,), vdt),
            pltpu.VMEM_SHARED((_N, d), vdt),
            [pltpu.SemaphoreType.DMA] * nb,
            [pltpu.SemaphoreType.DMA] * nb,
        ],
    )
    def spmm(rows_hbm, cols_hbm, w_hbm, x_hbm, b_hbm, out_hbm,
             rowv, colv, wv, rbufs, sbufs, bv, acc, gsems, ssems):
        src = x_hbm
        obufs = sbufs if dual_src else rbufs
        zbuf = obufs[0]
        c = lax.axis_index("c")
        s = lax.axis_index("s")
        wid = c * _NS + s

        def zrow(i, carry):
            for jj in range(d // vl):
                zbuf[i, pl.ds(jj * vl, vl)] = jnp.zeros((vl,), vdt)
            return carry
        lax.fori_loop(0, zr, zrow, 0)

        def zchunk(t, carry):
            g = s + t * _NS

            @pl.when(g < nzc)
            def _():
                pltpu.sync_copy(zbuf, acc.at[pl.ds(g * zr, zr)])
            return carry
        lax.fori_loop(0, npers, zchunk, 0)
        plsc.subcore_barrier()

        pltpu.sync_copy(rows_hbm.at[pl.ds(wid * ew, ew)], rowv)
        pltpu.sync_copy(cols_hbm.at[pl.ds(wid * ew, ew)], colv)
        pltpu.sync_copy(w_hbm.at[pl.ds(wid * ew, ew)], wv)
        pltpu.sync_copy(b_hbm, bv.at[pl.ds(0, b_hbm.shape[0])])
        if dual_src:
            blo = bv[pl.ds(0, vl)]
            bhi = bv[pl.ds(vl, vl)]

        def g_start(g, b):
            pltpu.async_copy(
                src.at[colv.at[pl.ds(g * chunk, chunk)]], rbufs[b],
                gsems[b])

        def g_wait(g, b):
            pltpu.make_async_copy(
                src.at[colv.at[pl.ds(g * chunk, chunk)]], rbufs[b],
                gsems[b]).wait()

        def s_start(g, b):
            pltpu.async_copy(
                obufs[b], acc.at[rowv.at[pl.ds(g * chunk, chunk)]],
                ssems[b], add=True)

        def s_wait(g, b):
            pltpu.make_async_copy(
                obufs[b], acc.at[rowv.at[pl.ds(g * chunk, chunk)]],
                ssems[b]).wait()

        def scale(g, b):
            rbuf = rbufs[b]
            obuf = obufs[b]

            @plsc.parallel_loop(0, chunk, step=1, unroll=unroll)
            def _(e):
                w16 = plsc.load_gather(
                    wv, [jnp.full((_L,), g * chunk + e, jnp.int32)])
                wvec = (plsc.pack(w16, w16, format=plsc.PackFormat.INTERLEAVED)
                        if use_bf16 else w16)
                for jj in range(d // vl):
                    sl = pl.ds(jj * vl, vl)
                    if dual_src:
                        hi = pl.ds(d + jj * vl, vl)
                        bias = (blo, bhi)[jj]
                        obuf[e, sl] = (rbuf[e, sl] + rbuf[e, hi]
                                       + bias) * wvec
                    else:
                        rbuf[e, sl] = rbuf[e, sl] * wvec

        for b0 in range(la):
            g_start(b0, b0 % nb)

        def chunk_body(gg, carry):
            for b_off in range(nb):
                g = gg * nb + b_off
                b = b_off

                @pl.when(g < nchunks)
                def _():
                    gn = g + la
                    bn = (b + la) % nb

                    @pl.when(gn < nchunks)
                    def _():
                        prev = gn - nb

                        @pl.when(prev >= 0)
                        def _():
                            s_wait(prev, bn)
                        g_start(gn, bn)
                    g_wait(g, b)
                    scale(g, b)
                    s_start(g, b)
            return carry
        lax.fori_loop(0, -(-nchunks // nb), chunk_body, 0)
        for k in range(1, min(nb, nchunks) + 1):
            gl = nchunks - k
            s_wait(gl, gl % nb)
        plsc.subcore_barrier()

        def ochunk(t, carry):
            g = s + t * _NS

            @pl.when(g < nzc)
            def _():
                pltpu.sync_copy(acc.at[pl.ds(g * zr, zr)], zbuf)
                if paired_out:
                    pltpu.sync_copy(zbuf, out_hbm.at[pl.ds(g * zr, zr), c])
                else:
                    pltpu.sync_copy(zbuf, out_hbm.at[c].at[pl.ds(g * zr, zr)])
            return carry
        lax.fori_loop(0, npers, ochunk, 0)

    return spmm


_TC_ROWS = 1000


def _tc_matmul(x, w, out_dtype=jnp.float32):
    n, din = x.shape
    dout = w.shape[1]
    def body(x_ref, w_ref, o_ref):
        o_ref[...] = jnp.dot(x_ref[...], w_ref[...],
                             preferred_element_type=jnp.float32
                             ).astype(out_dtype)
    return pl.pallas_call(
        body,
        grid=(n // _TC_ROWS,),
        in_specs=[pl.BlockSpec((_TC_ROWS, din), lambda i: (i, 0)),
                  pl.BlockSpec((din, dout), lambda i: (0, 0))],
        out_specs=pl.BlockSpec((_TC_ROWS, dout), lambda i: (i, 0)),
        out_shape=jax.ShapeDtypeStruct((n, dout), out_dtype),
    )(x, w)


def _tc_fuse2mm(s, w1, b, w2, out_dtype=jnp.float32):
    n, dh = s.shape[1], s.shape[2]
    dmid = w1.shape[1]
    dout = w2.shape[1]
    def body(s_ref, w1_ref, b_ref, w2_ref, o_ref):
        ssum = s_ref[0].astype(jnp.float32) + s_ref[1].astype(jnp.float32)
        h = jnp.dot(ssum, w1_ref[...], preferred_element_type=jnp.float32)
        h = jnp.maximum(h + b_ref[...], 0.0)
        o_ref[...] = jnp.dot(h, w2_ref[...],
                             preferred_element_type=jnp.float32
                             ).astype(out_dtype)
    return pl.pallas_call(
        body,
        grid=(n // _TC_ROWS,),
        in_specs=[pl.BlockSpec((2, _TC_ROWS, dh), lambda i: (0, i, 0)),
                  pl.BlockSpec((dh, dmid), lambda i: (0, 0)),
                  pl.BlockSpec((1, dmid), lambda i: (0, 0)),
                  pl.BlockSpec((dmid, dout), lambda i: (0, 0))],
        out_specs=pl.BlockSpec((_TC_ROWS, dout), lambda i: (i, 0)),
        out_shape=jax.ShapeDtypeStruct((n, dout), out_dtype),
    )(s, w1, b.reshape(1, dmid), w2)


def _tc_logsoftmax(s):
    n, dh = s.shape[1], s.shape[2]
    def body(s_ref, o_ref):
        o = s_ref[0].astype(jnp.float32) + s_ref[1].astype(jnp.float32)
        m = jnp.max(o, axis=1, keepdims=True)
        ex = jnp.exp(o - m)
        o_ref[...] = (o - m) - jnp.log(jnp.sum(ex, axis=1, keepdims=True))
    return pl.pallas_call(
        body,
        grid=(n // _TC_ROWS,),
        in_specs=[pl.BlockSpec((2, _TC_ROWS, dh), lambda i: (0, i, 0))],
        out_specs=pl.BlockSpec((_TC_ROWS, dh), lambda i: (i, 0)),
        out_shape=jax.ShapeDtypeStruct((n, dh), jnp.float32),
    )(s)


def kernel(x, adj_index, adj_weight, pvt_index, pvt_weight, W1, b1, W2, b2):
    e_a = adj_weight.shape[0]
    e_p = pvt_weight.shape[0]
    rows_a = adj_index[0].astype(jnp.int32)
    cols_a = adj_index[1].astype(jnp.int32)
    rows_p = pvt_index[0].astype(jnp.int32)
    cols_p = pvt_index[1].astype(jnp.int32)

    bf = jnp.bfloat16
    bdummy = jnp.zeros((8,), bf)
    xbf = x.astype(bf)
    s1 = _make_spmm(e_a, xbf.shape[1], 40, use_bf16=True)(
        rows_a, cols_a, adj_weight, xbf, bdummy)
    hw2 = _tc_fuse2mm(s1, W1, b1, W2, out_dtype=bf)
    s2 = _make_spmm(e_a, hw2.shape[1], 80, use_bf16=True,
                    paired_out=True)(
        rows_a, cols_a, adj_weight, hw2, bdummy)
    s3 = _make_spmm(e_p, 64, 40, use_bf16=True, dual_src=True)(
        rows_p, cols_p, pvt_weight, s2.reshape(_N, 128),
        b2.astype(bf))
    return _tc_logsoftmax(s3)


# speedup vs baseline: 1.1515x; 1.1515x over previous
import functools

import jax
import jax.numpy as jnp
from jax import lax
from jax.experimental import pallas as pl
from jax.experimental.pallas import tpu as pltpu
from jax.experimental.pallas import tpu_sc as plsc

_N = 10000
_NC = 2
_NS = 16
_NW = _NC * _NS
_L = 16


@functools.lru_cache(maxsize=None)
def _make_spmm(n_edges: int, d: int, chunk: int, use_bf16: bool = False,
               paired_out: bool = False, dual_src: bool = False):
    ew = n_edges // _NW
    assert ew * _NW == n_edges and ew % chunk == 0
    assert chunk % 8 == 0 and chunk <= 128 and d % _L == 0
    vdt = jnp.bfloat16 if use_bf16 else jnp.float32
    vl = 2 * _L if use_bf16 else _L
    dsrc = 2 * d if dual_src else d
    assert not dual_src or (use_bf16 and d == 2 * vl)
    nchunks = ew // chunk
    zr = chunk
    nzc = _N // zr
    npers = -(-nzc // _NS)
    assert _N % zr == 0
    unroll = 8
    nb = 6
    la = 3
    assert nchunks >= nb
    assert chunk % unroll == 0
    mesh = plsc.VectorSubcoreMesh(
        core_axis_name="c", subcore_axis_name="s",
        num_cores=_NC, num_subcores=_NS)

    out_shape = (_N, _NC, d) if paired_out else (_NC, _N, d)

    @functools.partial(
        pl.kernel,
        out_type=jax.ShapeDtypeStruct(out_shape, vdt),
        mesh=mesh,
        compiler_params=pltpu.CompilerParams(
            needs_layout_passes=False, use_tc_tiling_on_sc=False),
        scratch_types=[
            pltpu.VMEM((ew,), jnp.int32),
            pltpu.VMEM((ew,), jnp.int32),
            pltpu.VMEM((ew,), jnp.float32),
            [pltpu.VMEM((chunk, dsrc), vdt)] * nb,
            ([pltpu.VMEM((chunk, d), vdt)] * nb
             if dual_src else pltpu.SMEM((1,), jnp.int32)),
            pltpu.VMEM((64,), vdt),
            pltpu.VMEM_SHARED((_N, d), vdt),
            [pltpu.SemaphoreType.DMA] * nb,
            [pltpu.SemaphoreType.DMA] * nb,
        ],
    )
    def spmm(rows_hbm, cols_hbm, w_hbm, x_hbm, b_hbm, out_hbm,
             rowv, colv, wv, rbufs, sbufs, bv, acc, gsems, ssems):
        src = x_hbm
        obufs = sbufs if dual_src else rbufs
        zbuf = obufs[0]
        c = lax.axis_index("c")
        s = lax.axis_index("s")
        wid = c * _NS + s

        def zrow(i, carry):
            for jj in range(d // vl):
                zbuf[i, pl.ds(jj * vl, vl)] = jnp.zeros((vl,), vdt)
            return carry
        lax.fori_loop(0, zr, zrow, 0)

        def zchunk(t, carry):
            g = s + t * _NS

            @pl.when(g < nzc)
            def _():
                pltpu.sync_copy(zbuf, acc.at[pl.ds(g * zr, zr)])
            return carry
        lax.fori_loop(0, npers, zchunk, 0)
        plsc.subcore_barrier()

        pltpu.sync_copy(rows_hbm.at[pl.ds(wid * ew, ew)], rowv)
        pltpu.sync_copy(cols_hbm.at[pl.ds(wid * ew, ew)], colv)
        pltpu.sync_copy(w_hbm.at[pl.ds(wid * ew, ew)], wv)
        pltpu.sync_copy(b_hbm, bv.at[pl.ds(0, b_hbm.shape[0])])
        if dual_src:
            blo = bv[pl.ds(0, vl)]
            bhi = bv[pl.ds(vl, vl)]

        def g_start(g, b):
            pltpu.async_copy(
                src.at[colv.at[pl.ds(g * chunk, chunk)]], rbufs[b],
                gsems[b])

        def g_wait(g, b):
            pltpu.make_async_copy(
                src.at[colv.at[pl.ds(g * chunk, chunk)]], rbufs[b],
                gsems[b]).wait()

        def s_start(g, b):
            pltpu.async_copy(
                obufs[b], acc.at[rowv.at[pl.ds(g * chunk, chunk)]],
                ssems[b], add=True)

        def s_wait(g, b):
            pltpu.make_async_copy(
                obufs[b], acc.at[rowv.at[pl.ds(g * chunk, chunk)]],
                ssems[b]).wait()

        def scale(g, b):
            rbuf = rbufs[b]
            obuf = obufs[b]

            @plsc.parallel_loop(0, chunk, step=1, unroll=unroll)
            def _(e):
                w16 = plsc.load_gather(
                    wv, [jnp.full((_L,), g * chunk + e, jnp.int32)])
                wvec = (plsc.pack(w16, w16, format=plsc.PackFormat.INTERLEAVED)
                        if use_bf16 else w16)
                for jj in range(d // vl):
                    sl = pl.ds(jj * vl, vl)
                    if dual_src:
                        hi = pl.ds(d + jj * vl, vl)
                        bias = (blo, bhi)[jj]
                        obuf[e, sl] = (rbuf[e, sl] + rbuf[e, hi]
                                       + bias) * wvec
                    else:
                        rbuf[e, sl] = rbuf[e, sl] * wvec

        for b0 in range(la):
            g_start(b0, b0 % nb)

        def chunk_body(gg, carry):
            for b_off in range(nb):
                g = gg * nb + b_off
                b = b_off

                @pl.when(g < nchunks)
                def _():
                    gn = g + la
                    bn = (b + la) % nb

                    @pl.when(gn < nchunks)
                    def _():
                        prev = gn - nb

                        @pl.when(prev >= 0)
                        def _():
                            s_wait(prev, bn)
                        g_start(gn, bn)
                    g_wait(g, b)
                    scale(g, b)
                    s_start(g, b)
            return carry
        lax.fori_loop(0, -(-nchunks // nb), chunk_body, 0)
        for k in range(1, min(nb, nchunks) + 1):
            gl = nchunks - k
            s_wait(gl, gl % nb)
        plsc.subcore_barrier()

        def ochunk(t, carry):
            g = s + t * _NS

            @pl.when(g < nzc)
            def _():
                pltpu.sync_copy(acc.at[pl.ds(g * zr, zr)], zbuf)
                if paired_out:
                    pltpu.sync_copy(zbuf, out_hbm.at[pl.ds(g * zr, zr), c])
                else:
                    pltpu.sync_copy(zbuf, out_hbm.at[c].at[pl.ds(g * zr, zr)])
            return carry
        lax.fori_loop(0, npers, ochunk, 0)

    return spmm


_TC_ROWS = 1000


def _tc_matmul(x, w, out_dtype=jnp.float32):
    n, din = x.shape
    dout = w.shape[1]
    def body(x_ref, w_ref, o_ref):
        o_ref[...] = jnp.dot(x_ref[...], w_ref[...],
                             preferred_element_type=jnp.float32
                             ).astype(out_dtype)
    return pl.pallas_call(
        body,
        grid=(n // _TC_ROWS,),
        in_specs=[pl.BlockSpec((_TC_ROWS, din), lambda i: (i, 0)),
                  pl.BlockSpec((din, dout), lambda i: (0, 0))],
        out_specs=pl.BlockSpec((_TC_ROWS, dout), lambda i: (i, 0)),
        out_shape=jax.ShapeDtypeStruct((n, dout), out_dtype),
    )(x, w)


def _tc_fuse_relu_mm(s, b, w, out_dtype=jnp.float32):
    n, dh = s.shape[1], s.shape[2]
    dout = w.shape[1]
    def body(s_ref, b_ref, w_ref, o_ref):
        ssum = (s_ref[0].astype(jnp.float32) + s_ref[1].astype(jnp.float32)
                + b_ref[...])
        h = jnp.maximum(ssum, 0.0)
        o_ref[...] = jnp.dot(h, w_ref[...], preferred_element_type=jnp.float32
                             ).astype(out_dtype)
    return pl.pallas_call(
        body,
        grid=(n // _TC_ROWS,),
        in_specs=[pl.BlockSpec((2, _TC_ROWS, dh), lambda i: (0, i, 0)),
                  pl.BlockSpec((1, dh), lambda i: (0, 0)),
                  pl.BlockSpec((dh, dout), lambda i: (0, 0))],
        out_specs=pl.BlockSpec((_TC_ROWS, dout), lambda i: (i, 0)),
        out_shape=jax.ShapeDtypeStruct((n, dout), out_dtype),
    )(s, b.reshape(1, dh), w)


def _tc_add_bias(s, b, out_dtype=jnp.float32):
    n, dh = s.shape[1], s.shape[2]
    def body(s_ref, b_ref, o_ref):
        o_ref[...] = (s_ref[0].astype(jnp.float32)
                      + s_ref[1].astype(jnp.float32)
                      + b_ref[...]).astype(out_dtype)
    return pl.pallas_call(
        body,
        grid=(n // _TC_ROWS,),
        in_specs=[pl.BlockSpec((2, _TC_ROWS, dh), lambda i: (0, i, 0)),
                  pl.BlockSpec((1, dh), lambda i: (0, 0))],
        out_specs=pl.BlockSpec((_TC_ROWS, dh), lambda i: (i, 0)),
        out_shape=jax.ShapeDtypeStruct((n, dh), out_dtype),
    )(s, b.reshape(1, dh))


def _tc_fuse2mm(s, w1, b, w2, out_dtype=jnp.float32):
    n, dh = s.shape[1], s.shape[2]
    dmid = w1.shape[1]
    dout = w2.shape[1]
    def body(s_ref, w1_ref, b_ref, w2_ref, o_ref):
        ssum = s_ref[0].astype(jnp.float32) + s_ref[1].astype(jnp.float32)
        h = jnp.dot(ssum, w1_ref[...], preferred_element_type=jnp.float32)
        h = jnp.maximum(h + b_ref[...], 0.0)
        o_ref[...] = jnp.dot(h, w2_ref[...],
                             preferred_element_type=jnp.float32
                             ).astype(out_dtype)
    return pl.pallas_call(
        body,
        grid=(n // _TC_ROWS,),
        in_specs=[pl.BlockSpec((2, _TC_ROWS, dh), lambda i: (0, i, 0)),
                  pl.BlockSpec((dh, dmid), lambda i: (0, 0)),
                  pl.BlockSpec((1, dmid), lambda i: (0, 0)),
                  pl.BlockSpec((dmid, dout), lambda i: (0, 0))],
        out_specs=pl.BlockSpec((_TC_ROWS, dout), lambda i: (i, 0)),
        out_shape=jax.ShapeDtypeStruct((n, dout), out_dtype),
    )(s, w1, b.reshape(1, dmid), w2)


def _tc_logsoftmax(s):
    n, dh = s.shape[1], s.shape[2]
    def body(s_ref, o_ref):
        o = s_ref[0].astype(jnp.float32) + s_ref[1].astype(jnp.float32)
        m = jnp.max(o, axis=1, keepdims=True)
        ex = jnp.exp(o - m)
        o_ref[...] = (o - m) - jnp.log(jnp.sum(ex, axis=1, keepdims=True))
    return pl.pallas_call(
        body,
        grid=(n // _TC_ROWS,),
        in_specs=[pl.BlockSpec((2, _TC_ROWS, dh), lambda i: (0, i, 0))],
        out_specs=pl.BlockSpec((_TC_ROWS, dh), lambda i: (i, 0)),
        out_shape=jax.ShapeDtypeStruct((n, dh), jnp.float32),
    )(s)


def kernel(x, adj_index, adj_weight, pvt_index, pvt_weight, W1, b1, W2, b2):
    e_a = adj_weight.shape[0]
    e_p = pvt_weight.shape[0]
    rows_a = adj_index[0].astype(jnp.int32)
    cols_a = adj_index[1].astype(jnp.int32)
    rows_p = pvt_index[0].astype(jnp.int32)
    cols_p = pvt_index[1].astype(jnp.int32)

    bf = jnp.bfloat16
    bdummy = jnp.zeros((8,), bf)
    xw1 = _tc_matmul(x, W1, out_dtype=bf)
    s1 = _make_spmm(e_a, xw1.shape[1], 40, use_bf16=True)(
        rows_a, cols_a, adj_weight, xw1, bdummy)
    hw2 = _tc_fuse_relu_mm(s1, b1, W2, out_dtype=bf)
    s2 = _make_spmm(e_a, hw2.shape[1], 80, use_bf16=True)(
        rows_a, cols_a, adj_weight, hw2, bdummy)
    h2 = _tc_add_bias(s2, b2, out_dtype=bf)
    s3 = _make_spmm(e_p, h2.shape[1], 40, use_bf16=True)(
        rows_p, cols_p, pvt_weight, h2, bdummy)
    return _tc_logsoftmax(s3)


# speedup vs baseline: 1.2048x; 1.0462x over previous
import functools

import jax
import jax.numpy as jnp
from jax import lax
from jax.experimental import pallas as pl
from jax.experimental.pallas import tpu as pltpu
from jax.experimental.pallas import tpu_sc as plsc

_N = 10000
_NC = 2
_NS = 16
_NW = _NC * _NS
_L = 16


@functools.lru_cache(maxsize=None)
def _make_spmm(n_edges: int, d: int, chunk: int, use_bf16: bool = False,
               paired_out: bool = False, dual_src: bool = False):
    ew = n_edges // _NW
    assert ew * _NW == n_edges and ew % chunk == 0
    assert chunk % 8 == 0 and chunk <= 128 and d % _L == 0
    vdt = jnp.bfloat16 if use_bf16 else jnp.float32
    vl = 2 * _L if use_bf16 else _L
    dsrc = 2 * d if dual_src else d
    assert not dual_src or (use_bf16 and d == 2 * vl)
    nchunks = ew // chunk
    zr = chunk
    nzc = _N // zr
    npers = -(-nzc // _NS)
    assert _N % zr == 0
    unroll = 8
    nb = 8
    la = 4
    assert nchunks >= nb
    assert chunk % unroll == 0
    mesh = plsc.VectorSubcoreMesh(
        core_axis_name="c", subcore_axis_name="s",
        num_cores=_NC, num_subcores=_NS)

    out_shape = (_N, _NC, d) if paired_out else (_NC, _N, d)

    @functools.partial(
        pl.kernel,
        out_type=jax.ShapeDtypeStruct(out_shape, vdt),
        mesh=mesh,
        compiler_params=pltpu.CompilerParams(
            needs_layout_passes=False, use_tc_tiling_on_sc=False),
        scratch_types=[
            pltpu.VMEM((ew,), jnp.int32),
            pltpu.VMEM((ew,), jnp.int32),
            pltpu.VMEM((ew,), jnp.float32),
            [pltpu.VMEM((chunk, dsrc), vdt)] * nb,
            ([pltpu.VMEM((chunk, d), vdt)] * nb
             if dual_src else pltpu.SMEM((1,), jnp.int32)),
            pltpu.VMEM((64,), vdt),
            pltpu.VMEM_SHARED((_N, d), vdt),
            [pltpu.SemaphoreType.DMA] * nb,
            [pltpu.SemaphoreType.DMA] * nb,
        ],
    )
    def spmm(rows_hbm, cols_hbm, w_hbm, x_hbm, b_hbm, out_hbm,
             rowv, colv, wv, rbufs, sbufs, bv, acc, gsems, ssems):
        src = x_hbm
        obufs = sbufs if dual_src else rbufs
        zbuf = obufs[0]
        c = lax.axis_index("c")
        s = lax.axis_index("s")
        wid = c * _NS + s

        def zrow(i, carry):
            for jj in range(d // vl):
                zbuf[i, pl.ds(jj * vl, vl)] = jnp.zeros((vl,), vdt)
            return carry
        lax.fori_loop(0, zr, zrow, 0)

        def zchunk(t, carry):
            g = s + t * _NS

            @pl.when(g < nzc)
            def _():
                pltpu.sync_copy(zbuf, acc.at[pl.ds(g * zr, zr)])
            return carry
        lax.fori_loop(0, npers, zchunk, 0)
        plsc.subcore_barrier()

        pltpu.sync_copy(rows_hbm.at[pl.ds(wid * ew, ew)], rowv)
        pltpu.sync_copy(cols_hbm.at[pl.ds(wid * ew, ew)], colv)
        pltpu.sync_copy(w_hbm.at[pl.ds(wid * ew, ew)], wv)
        pltpu.sync_copy(b_hbm, bv.at[pl.ds(0, b_hbm.shape[0])])
        if dual_src:
            blo = bv[pl.ds(0, vl)]
            bhi = bv[pl.ds(vl, vl)]

        def g_start(g, b):
            pltpu.async_copy(
                src.at[colv.at[pl.ds(g * chunk, chunk)]], rbufs[b],
                gsems[b])

        def g_wait(g, b):
            pltpu.make_async_copy(
                src.at[colv.at[pl.ds(g * chunk, chunk)]], rbufs[b],
                gsems[b]).wait()

        def s_start(g, b):
            pltpu.async_copy(
                obufs[b], acc.at[rowv.at[pl.ds(g * chunk, chunk)]],
                ssems[b], add=True)

        def s_wait(g, b):
            pltpu.make_async_copy(
                obufs[b], acc.at[rowv.at[pl.ds(g * chunk, chunk)]],
                ssems[b]).wait()

        def scale(g, b):
            rbuf = rbufs[b]
            obuf = obufs[b]

            @plsc.parallel_loop(0, chunk, step=1, unroll=unroll)
            def _(e):
                w16 = plsc.load_gather(
                    wv, [jnp.full((_L,), g * chunk + e, jnp.int32)])
                wvec = (plsc.pack(w16, w16, format=plsc.PackFormat.INTERLEAVED)
                        if use_bf16 else w16)
                for jj in range(d // vl):
                    sl = pl.ds(jj * vl, vl)
                    if dual_src:
                        hi = pl.ds(d + jj * vl, vl)
                        bias = (blo, bhi)[jj]
                        obuf[e, sl] = (rbuf[e, sl] + rbuf[e, hi]
                                       + bias) * wvec
                    else:
                        rbuf[e, sl] = rbuf[e, sl] * wvec

        for b0 in range(la):
            g_start(b0, b0 % nb)

        def chunk_body(gg, carry):
            for b_off in range(nb):
                g = gg * nb + b_off
                b = b_off

                @pl.when(g < nchunks)
                def _():
                    gn = g + la
                    bn = (b + la) % nb

                    @pl.when(gn < nchunks)
                    def _():
                        prev = gn - nb

                        @pl.when(prev >= 0)
                        def _():
                            s_wait(prev, bn)
                        g_start(gn, bn)
                    g_wait(g, b)
                    scale(g, b)
                    s_start(g, b)
            return carry
        lax.fori_loop(0, -(-nchunks // nb), chunk_body, 0)
        for k in range(1, min(nb, nchunks) + 1):
            gl = nchunks - k
            s_wait(gl, gl % nb)
        plsc.subcore_barrier()

        def ochunk(t, carry):
            g = s + t * _NS

            @pl.when(g < nzc)
            def _():
                pltpu.sync_copy(acc.at[pl.ds(g * zr, zr)], zbuf)
                if paired_out:
                    pltpu.sync_copy(zbuf, out_hbm.at[pl.ds(g * zr, zr), c])
                else:
                    pltpu.sync_copy(zbuf, out_hbm.at[c].at[pl.ds(g * zr, zr)])
            return carry
        lax.fori_loop(0, npers, ochunk, 0)

    return spmm


_TC_ROWS = 1000


def _tc_matmul(x, w, out_dtype=jnp.float32):
    n, din = x.shape
    dout = w.shape[1]
    def body(x_ref, w_ref, o_ref):
        o_ref[...] = jnp.dot(x_ref[...], w_ref[...],
                             preferred_element_type=jnp.float32
                             ).astype(out_dtype)
    return pl.pallas_call(
        body,
        grid=(n // _TC_ROWS,),
        in_specs=[pl.BlockSpec((_TC_ROWS, din), lambda i: (i, 0)),
                  pl.BlockSpec((din, dout), lambda i: (0, 0))],
        out_specs=pl.BlockSpec((_TC_ROWS, dout), lambda i: (i, 0)),
        out_shape=jax.ShapeDtypeStruct((n, dout), out_dtype),
    )(x, w)


def _tc_fuse_relu_mm(s, b, w, out_dtype=jnp.float32):
    n, dh = s.shape[1], s.shape[2]
    dout = w.shape[1]
    def body(s_ref, b_ref, w_ref, o_ref):
        ssum = (s_ref[0].astype(jnp.float32) + s_ref[1].astype(jnp.float32)
                + b_ref[...])
        h = jnp.maximum(ssum, 0.0)
        o_ref[...] = jnp.dot(h, w_ref[...], preferred_element_type=jnp.float32
                             ).astype(out_dtype)
    return pl.pallas_call(
        body,
        grid=(n // _TC_ROWS,),
        in_specs=[pl.BlockSpec((2, _TC_ROWS, dh), lambda i: (0, i, 0)),
                  pl.BlockSpec((1, dh), lambda i: (0, 0)),
                  pl.BlockSpec((dh, dout), lambda i: (0, 0))],
        out_specs=pl.BlockSpec((_TC_ROWS, dout), lambda i: (i, 0)),
        out_shape=jax.ShapeDtypeStruct((n, dout), out_dtype),
    )(s, b.reshape(1, dh), w)


def _tc_add_bias(s, b, out_dtype=jnp.float32):
    n, dh = s.shape[1], s.shape[2]
    def body(s_ref, b_ref, o_ref):
        o_ref[...] = (s_ref[0].astype(jnp.float32)
                      + s_ref[1].astype(jnp.float32)
                      + b_ref[...]).astype(out_dtype)
    return pl.pallas_call(
        body,
        grid=(n // _TC_ROWS,),
        in_specs=[pl.BlockSpec((2, _TC_ROWS, dh), lambda i: (0, i, 0)),
                  pl.BlockSpec((1, dh), lambda i: (0, 0))],
        out_specs=pl.BlockSpec((_TC_ROWS, dh), lambda i: (i, 0)),
        out_shape=jax.ShapeDtypeStruct((n, dh), out_dtype),
    )(s, b.reshape(1, dh))


def _tc_fuse2mm(s, w1, b, w2, out_dtype=jnp.float32):
    n, dh = s.shape[1], s.shape[2]
    dmid = w1.shape[1]
    dout = w2.shape[1]
    def body(s_ref, w1_ref, b_ref, w2_ref, o_ref):
        ssum = s_ref[0].astype(jnp.float32) + s_ref[1].astype(jnp.float32)
        h = jnp.dot(ssum, w1_ref[...], preferred_element_type=jnp.float32)
        h = jnp.maximum(h + b_ref[...], 0.0)
        o_ref[...] = jnp.dot(h, w2_ref[...],
                             preferred_element_type=jnp.float32
                             ).astype(out_dtype)
    return pl.pallas_call(
        body,
        grid=(n // _TC_ROWS,),
        in_specs=[pl.BlockSpec((2, _TC_ROWS, dh), lambda i: (0, i, 0)),
                  pl.BlockSpec((dh, dmid), lambda i: (0, 0)),
                  pl.BlockSpec((1, dmid), lambda i: (0, 0)),
                  pl.BlockSpec((dmid, dout), lambda i: (0, 0))],
        out_specs=pl.BlockSpec((_TC_ROWS, dout), lambda i: (i, 0)),
        out_shape=jax.ShapeDtypeStruct((n, dout), out_dtype),
    )(s, w1, b.reshape(1, dmid), w2)


def _tc_logsoftmax(s):
    n, dh = s.shape[1], s.shape[2]
    def body(s_ref, o_ref):
        o = s_ref[0].astype(jnp.float32) + s_ref[1].astype(jnp.float32)
        m = jnp.max(o, axis=1, keepdims=True)
        ex = jnp.exp(o - m)
        o_ref[...] = (o - m) - jnp.log(jnp.sum(ex, axis=1, keepdims=True))
    return pl.pallas_call(
        body,
        grid=(n // _TC_ROWS,),
        in_specs=[pl.BlockSpec((2, _TC_ROWS, dh), lambda i: (0, i, 0))],
        out_specs=pl.BlockSpec((_TC_ROWS, dh), lambda i: (i, 0)),
        out_shape=jax.ShapeDtypeStruct((n, dh), jnp.float32),
    )(s)


def kernel(x, adj_index, adj_weight, pvt_index, pvt_weight, W1, b1, W2, b2):
    e_a = adj_weight.shape[0]
    e_p = pvt_weight.shape[0]
    rows_a = adj_index[0].astype(jnp.int32)
    cols_a = adj_index[1].astype(jnp.int32)
    rows_p = pvt_index[0].astype(jnp.int32)
    cols_p = pvt_index[1].astype(jnp.int32)

    bf = jnp.bfloat16
    bdummy = jnp.zeros((8,), bf)
    xw1 = _tc_matmul(x, W1, out_dtype=bf)
    s1 = _make_spmm(e_a, xw1.shape[1], 40, use_bf16=True)(
        rows_a, cols_a, adj_weight, xw1, bdummy)
    hw2 = _tc_fuse_relu_mm(s1, b1, W2, out_dtype=bf)
    s2 = _make_spmm(e_a, hw2.shape[1], 80, use_bf16=True)(
        rows_a, cols_a, adj_weight, hw2, bdummy)
    h2 = _tc_add_bias(s2, b2, out_dtype=bf)
    s3 = _make_spmm(e_p, h2.shape[1], 40, use_bf16=True)(
        rows_p, cols_p, pvt_weight, h2, bdummy)
    return _tc_logsoftmax(s3)
